# baseline (speedup 1.0000x reference)
import jax
import jax.numpy as jnp
from jax import lax
from jax.experimental import pallas as pl
from jax.experimental.pallas import tpu as pltpu
from jax.experimental.pallas import tpu_sc as plsc

_D = 64
_NC, _NS = 2, 16
_NW = _NC * _NS
_CHUNK = 800
_BC = 2048
_H = _BC // 2
_SHIFT = _H.bit_length() - 1


def _transpose_body(wt_ref, wp_ref):
    t = wt_ref[...].T
    wp_ref[:, :_D] = t[:_H, :]
    wp_ref[:, _D:] = t[_H:, :]


def _transpose_table(wt):
    nb = pl.cdiv(wt.shape[1], _BC)
    return pl.pallas_call(
        _transpose_body,
        grid=(nb,),
        in_specs=[pl.BlockSpec((_D, _BC), lambda i: (0, i))],
        out_specs=pl.BlockSpec((_H, 2 * _D), lambda i: (i, 0)),
        out_shape=jax.ShapeDtypeStruct((nb * _H, 2 * _D), jnp.float32),
    )(wt)


def _gather_body(idx_hbm, table_hbm, out_hbm,
                 idx_v, rows0, rows1, gsem0, gsem1, ssem0, ssem1):
    wid = lax.axis_index("s") * _NC + lax.axis_index("c")
    bpw = idx_v.shape[0]
    base = wid * bpw
    nchunk = bpw // _CHUNK
    rows = (rows0, rows1)
    gsem = (gsem0, gsem1)
    ssem = (ssem0, ssem1)

    pltpu.sync_copy(idx_hbm.at[pl.ds(base, bpw)], idx_v)

    @pl.loop(0, bpw // 16, unroll=8)
    def _xform(i):
        v = idx_v[pl.ds(i * 16, 16)]
        j = lax.bitwise_and(v, _BC - 1)
        hi = lax.shift_right_logical(j, _SHIFT)
        idx_v[pl.ds(i * 16, 16)] = v + j - (_BC - 1) * hi

    def start_gather(c, b):
        pltpu.async_copy(
            table_hbm.at[idx_v.at[pl.ds(c * _CHUNK, _CHUNK)]], rows[b], gsem[b]
        )

    start_gather(0, 0)
    start_gather(1, 1)

    @pl.loop(0, nchunk, step=2)
    def _pair(g):
        for b in range(2):
            c = g + b
            o = 1 - b
            @pl.when(jnp.logical_and(c >= 1, c + 1 < nchunk))
            def _refill():
                pltpu.make_async_copy(
                    rows[o], out_hbm.at[pl.ds(0, _CHUNK), pl.ds(0, _D)], ssem[o]
                ).wait()
                start_gather(c + 1, o)

            pltpu.make_async_copy(
                table_hbm.at[idx_v.at[pl.ds(0, _CHUNK)]], rows[b], gsem[b]
            ).wait()
            pltpu.async_copy(
                rows[b],
                out_hbm.at[pl.ds(base + c * _CHUNK, _CHUNK), pl.ds(0, _D)],
                ssem[b],
            )

    for b in range(2):
        pltpu.make_async_copy(
            rows[b], out_hbm.at[pl.ds(0, _CHUNK), pl.ds(0, _D)], ssem[b]
        ).wait()


def kernel(input, weight):
    b = input.size
    flat_idx = input.reshape(b).astype(jnp.int32)
    wp = _transpose_table(weight.T)
    wd = wp.reshape(2 * wp.shape[0], _D)
    bpw = b // _NW
    k = pl.kernel(
        _gather_body,
        out_type=jax.ShapeDtypeStruct((b, 2 * _D), jnp.float32),
        mesh=plsc.VectorSubcoreMesh(core_axis_name="c", subcore_axis_name="s"),
        scratch_types=[
            pltpu.VMEM((bpw,), jnp.int32),
            pltpu.VMEM((_CHUNK, _D), jnp.float32),
            pltpu.VMEM((_CHUNK, _D), jnp.float32),
            pltpu.SemaphoreType.DMA,
            pltpu.SemaphoreType.DMA,
            pltpu.SemaphoreType.DMA,
            pltpu.SemaphoreType.DMA,
        ],
        compiler_params=pltpu.CompilerParams(use_tc_tiling_on_sc=False),
    )
    out = k(flat_idx, wd)
    return out[:, :_D].reshape(*input.shape, _D)


# speedup vs baseline: 1.3005x; 1.3005x over previous
import jax
import jax.numpy as jnp
from jax import lax
from jax.experimental import pallas as pl
from jax.experimental.pallas import tpu as pltpu
from jax.experimental.pallas import tpu_sc as plsc

_D = 64
_NC, _NS = 2, 16
_NW = _NC * _NS
_CHUNK = 800
_BC = 8192
_H = _BC // 2
_SHIFT = _H.bit_length() - 1


def _transpose_body(wt_ref, wp_ref):
    t = wt_ref[...].T
    wp_ref[:, :_D] = t[:_H, :]
    wp_ref[:, _D:] = t[_H:, :]


def _transpose_table(wt):
    nb = pl.cdiv(wt.shape[1], _BC)
    return pl.pallas_call(
        _transpose_body,
        grid=(nb,),
        in_specs=[pl.BlockSpec((_D, _BC), lambda i: (0, i))],
        out_specs=pl.BlockSpec((_H, 2 * _D), lambda i: (i, 0)),
        out_shape=jax.ShapeDtypeStruct((nb * _H, 2 * _D), jnp.float32),
    )(wt)


def _gather_body(idx_hbm, table_hbm, out_hbm,
                 idx_v, rows0, rows1, gsem0, gsem1, ssem0, ssem1):
    wid = lax.axis_index("s") * _NC + lax.axis_index("c")
    bpw = idx_v.shape[0]
    base = wid * bpw
    nchunk = bpw // _CHUNK
    rows = (rows0, rows1)
    gsem = (gsem0, gsem1)
    ssem = (ssem0, ssem1)

    pltpu.sync_copy(idx_hbm.at[pl.ds(base, bpw)], idx_v)

    @pl.loop(0, bpw // 16, unroll=8)
    def _xform(i):
        v = idx_v[pl.ds(i * 16, 16)]
        j = lax.bitwise_and(v, _BC - 1)
        hi = lax.shift_right_logical(j, _SHIFT)
        idx_v[pl.ds(i * 16, 16)] = v + j - (_BC - 1) * hi

    def start_gather(c, b):
        pltpu.async_copy(
            table_hbm.at[idx_v.at[pl.ds(c * _CHUNK, _CHUNK)]], rows[b], gsem[b]
        )

    start_gather(0, 0)
    start_gather(1, 1)

    @pl.loop(0, nchunk, step=2)
    def _pair(g):
        for b in range(2):
            c = g + b
            o = 1 - b
            @pl.when(jnp.logical_and(c >= 1, c + 1 < nchunk))
            def _refill():
                pltpu.make_async_copy(
                    rows[o], out_hbm.at[pl.ds(0, _CHUNK), pl.ds(0, _D)], ssem[o]
                ).wait()
                start_gather(c + 1, o)

            pltpu.make_async_copy(
                table_hbm.at[idx_v.at[pl.ds(0, _CHUNK)]], rows[b], gsem[b]
            ).wait()
            pltpu.async_copy(
                rows[b],
                out_hbm.at[pl.ds(base + c * _CHUNK, _CHUNK), pl.ds(0, _D)],
                ssem[b],
            )

    for b in range(2):
        pltpu.make_async_copy(
            rows[b], out_hbm.at[pl.ds(0, _CHUNK), pl.ds(0, _D)], ssem[b]
        ).wait()


def kernel(input, weight):
    b = input.size
    flat_idx = input.reshape(b).astype(jnp.int32)
    wp = _transpose_table(weight.T)
    wd = wp.reshape(2 * wp.shape[0], _D)
    bpw = b // _NW
    k = pl.kernel(
        _gather_body,
        out_type=jax.ShapeDtypeStruct((b, 2 * _D), jnp.float32),
        mesh=plsc.VectorSubcoreMesh(core_axis_name="c", subcore_axis_name="s"),
        scratch_types=[
            pltpu.VMEM((bpw,), jnp.int32),
            pltpu.VMEM((_CHUNK, _D), jnp.float32),
            pltpu.VMEM((_CHUNK, _D), jnp.float32),
            pltpu.SemaphoreType.DMA,
            pltpu.SemaphoreType.DMA,
            pltpu.SemaphoreType.DMA,
            pltpu.SemaphoreType.DMA,
        ],
        compiler_params=pltpu.CompilerParams(use_tc_tiling_on_sc=False),
    )
    out = k(flat_idx, wd)
    return out[:, :_D].reshape(*input.shape, _D)


# speedup vs baseline: 1.3859x; 1.0657x over previous
import jax
import jax.numpy as jnp
from jax import lax
from jax.experimental import pallas as pl
from jax.experimental.pallas import tpu as pltpu
from jax.experimental.pallas import tpu_sc as plsc

_D = 64
_NC, _NS = 2, 16
_NW = _NC * _NS
_CHUNK = 800
_BC = 16384
_H = _BC // 2
_SHIFT = _H.bit_length() - 1


def _transpose_body(wt_ref, wp_ref):
    t = wt_ref[...].T
    wp_ref[:, :_D] = t[:_H, :]
    wp_ref[:, _D:] = t[_H:, :]


def _transpose_table(wt):
    nb = pl.cdiv(wt.shape[1], _BC)
    return pl.pallas_call(
        _transpose_body,
        grid=(nb,),
        in_specs=[pl.BlockSpec((_D, _BC), lambda i: (0, i))],
        out_specs=pl.BlockSpec((_H, 2 * _D), lambda i: (i, 0)),
        out_shape=jax.ShapeDtypeStruct((nb * _H, 2 * _D), jnp.float32),
    )(wt)


def _gather_body(idx_hbm, table_hbm, out_hbm,
                 idx_v, rows0, rows1, gsem0, gsem1, ssem0, ssem1):
    wid = lax.axis_index("s") * _NC + lax.axis_index("c")
    bpw = idx_v.shape[0]
    base = wid * bpw
    nchunk = bpw // _CHUNK
    rows = (rows0, rows1)
    gsem = (gsem0, gsem1)
    ssem = (ssem0, ssem1)

    pltpu.sync_copy(idx_hbm.at[pl.ds(base, bpw)], idx_v)

    @pl.loop(0, bpw // 16, unroll=8)
    def _xform(i):
        v = idx_v[pl.ds(i * 16, 16)]
        j = lax.bitwise_and(v, _BC - 1)
        hi = lax.shift_right_logical(j, _SHIFT)
        idx_v[pl.ds(i * 16, 16)] = v + j - (_BC - 1) * hi

    def start_gather(c, b):
        pltpu.async_copy(
            table_hbm.at[idx_v.at[pl.ds(c * _CHUNK, _CHUNK)]], rows[b], gsem[b]
        )

    start_gather(0, 0)
    start_gather(1, 1)

    @pl.loop(0, nchunk, step=2)
    def _pair(g):
        for b in range(2):
            c = g + b
            o = 1 - b
            @pl.when(jnp.logical_and(c >= 1, c + 1 < nchunk))
            def _refill():
                pltpu.make_async_copy(
                    rows[o], out_hbm.at[pl.ds(0, _CHUNK), pl.ds(0, _D)], ssem[o]
                ).wait()
                start_gather(c + 1, o)

            pltpu.make_async_copy(
                table_hbm.at[idx_v.at[pl.ds(0, _CHUNK)]], rows[b], gsem[b]
            ).wait()
            pltpu.async_copy(
                rows[b],
                out_hbm.at[pl.ds(base + c * _CHUNK, _CHUNK), pl.ds(0, _D)],
                ssem[b],
            )

    for b in range(2):
        pltpu.make_async_copy(
            rows[b], out_hbm.at[pl.ds(0, _CHUNK), pl.ds(0, _D)], ssem[b]
        ).wait()


def kernel(input, weight):
    b = input.size
    flat_idx = input.reshape(b).astype(jnp.int32)
    wp = _transpose_table(weight.T)
    wd = wp.reshape(2 * wp.shape[0], _D)
    bpw = b // _NW
    k = pl.kernel(
        _gather_body,
        out_type=jax.ShapeDtypeStruct((b, 2 * _D), jnp.float32),
        mesh=plsc.VectorSubcoreMesh(core_axis_name="c", subcore_axis_name="s"),
        scratch_types=[
            pltpu.VMEM((bpw,), jnp.int32),
            pltpu.VMEM((_CHUNK, _D), jnp.float32),
            pltpu.VMEM((_CHUNK, _D), jnp.float32),
            pltpu.SemaphoreType.DMA,
            pltpu.SemaphoreType.DMA,
            pltpu.SemaphoreType.DMA,
            pltpu.SemaphoreType.DMA,
        ],
        compiler_params=pltpu.CompilerParams(use_tc_tiling_on_sc=False),
    )
    out = k(flat_idx, wd)
    return out[:, :_D].reshape(*input.shape, _D)


# speedup vs baseline: 1.4226x; 1.0265x over previous
import jax
import jax.numpy as jnp
from jax import lax
from jax.experimental import pallas as pl
from jax.experimental.pallas import tpu as pltpu
from jax.experimental.pallas import tpu_sc as plsc

_D = 64
_NC, _NS = 2, 16
_NW = _NC * _NS
_CHUNK = 800
_BC = 32768
_H = _BC // 2
_SHIFT = _H.bit_length() - 1


def _transpose_body(wt_ref, wp_ref):
    t = wt_ref[...].T
    wp_ref[:, :_D] = t[:_H, :]
    wp_ref[:, _D:] = t[_H:, :]


def _transpose_table(wt):
    nb = pl.cdiv(wt.shape[1], _BC)
    return pl.pallas_call(
        _transpose_body,
        grid=(nb,),
        in_specs=[pl.BlockSpec((_D, _BC), lambda i: (0, i))],
        out_specs=pl.BlockSpec((_H, 2 * _D), lambda i: (i, 0)),
        out_shape=jax.ShapeDtypeStruct((nb * _H, 2 * _D), jnp.float32),
    )(wt)


def _gather_body(idx_hbm, table_hbm, out_hbm,
                 idx_v, rows0, rows1, gsem0, gsem1, ssem0, ssem1):
    wid = lax.axis_index("s") * _NC + lax.axis_index("c")
    bpw = idx_v.shape[0]
    base = wid * bpw
    nchunk = bpw // _CHUNK
    rows = (rows0, rows1)
    gsem = (gsem0, gsem1)
    ssem = (ssem0, ssem1)

    pltpu.sync_copy(idx_hbm.at[pl.ds(base, bpw)], idx_v)

    @pl.loop(0, bpw // 16, unroll=8)
    def _xform(i):
        v = idx_v[pl.ds(i * 16, 16)]
        j = lax.bitwise_and(v, _BC - 1)
        hi = lax.shift_right_logical(j, _SHIFT)
        idx_v[pl.ds(i * 16, 16)] = v + j - (_BC - 1) * hi

    def start_gather(c, b):
        pltpu.async_copy(
            table_hbm.at[idx_v.at[pl.ds(c * _CHUNK, _CHUNK)]], rows[b], gsem[b]
        )

    start_gather(0, 0)
    start_gather(1, 1)

    @pl.loop(0, nchunk, step=2)
    def _pair(g):
        for b in range(2):
            c = g + b
            o = 1 - b
            @pl.when(jnp.logical_and(c >= 1, c + 1 < nchunk))
            def _refill():
                pltpu.make_async_copy(
                    rows[o], out_hbm.at[pl.ds(0, _CHUNK), pl.ds(0, _D)], ssem[o]
                ).wait()
                start_gather(c + 1, o)

            pltpu.make_async_copy(
                table_hbm.at[idx_v.at[pl.ds(0, _CHUNK)]], rows[b], gsem[b]
            ).wait()
            pltpu.async_copy(
                rows[b],
                out_hbm.at[pl.ds(base + c * _CHUNK, _CHUNK), pl.ds(0, _D)],
                ssem[b],
            )

    for b in range(2):
        pltpu.make_async_copy(
            rows[b], out_hbm.at[pl.ds(0, _CHUNK), pl.ds(0, _D)], ssem[b]
        ).wait()


def kernel(input, weight):
    b = input.size
    flat_idx = input.reshape(b).astype(jnp.int32)
    wp = _transpose_table(weight.T)
    wd = wp.reshape(2 * wp.shape[0], _D)
    bpw = b // _NW
    k = pl.kernel(
        _gather_body,
        out_type=jax.ShapeDtypeStruct((b, 2 * _D), jnp.float32),
        mesh=plsc.VectorSubcoreMesh(core_axis_name="c", subcore_axis_name="s"),
        scratch_types=[
            pltpu.VMEM((bpw,), jnp.int32),
            pltpu.VMEM((_CHUNK, _D), jnp.float32),
            pltpu.VMEM((_CHUNK, _D), jnp.float32),
            pltpu.SemaphoreType.DMA,
            pltpu.SemaphoreType.DMA,
            pltpu.SemaphoreType.DMA,
            pltpu.SemaphoreType.DMA,
        ],
        compiler_params=pltpu.CompilerParams(use_tc_tiling_on_sc=False),
    )
    out = k(flat_idx, wd)
    return out[:, :_D].reshape(*input.shape, _D)


# speedup vs baseline: 1.4228x; 1.0001x over previous
import jax
import jax.numpy as jnp
from jax import lax
from jax.experimental import pallas as pl
from jax.experimental.pallas import tpu as pltpu
from jax.experimental.pallas import tpu_sc as plsc

_D = 64
_NC, _NS = 2, 16
_NW = _NC * _NS
_CHUNK = 800
_BC = 32768
_H = _BC // 2
_SHIFT = _H.bit_length() - 1


_S = 2048


def _transpose_body(wt_ref, wp_ref):
    for s in range(_BC // _S):
        t = wt_ref[:, pl.ds(s * _S, _S)].T
        if s * _S < _H:
            wp_ref[pl.ds(s * _S, _S), : _D] = t
        else:
            wp_ref[pl.ds(s * _S - _H, _S), _D:] = t


def _transpose_table(wt):
    nb = pl.cdiv(wt.shape[1], _BC)
    return pl.pallas_call(
        _transpose_body,
        grid=(nb,),
        in_specs=[pl.BlockSpec((_D, _BC), lambda i: (0, i))],
        out_specs=pl.BlockSpec((_H, 2 * _D), lambda i: (i, 0)),
        out_shape=jax.ShapeDtypeStruct((nb * _H, 2 * _D), jnp.float32),
    )(wt)


def _gather_body(idx_hbm, table_hbm, out_hbm,
                 idx_v, rows0, rows1, gsem0, gsem1, ssem0, ssem1):
    wid = lax.axis_index("s") * _NC + lax.axis_index("c")
    bpw = idx_v.shape[0]
    base = wid * bpw
    nchunk = bpw // _CHUNK
    rows = (rows0, rows1)
    gsem = (gsem0, gsem1)
    ssem = (ssem0, ssem1)

    pltpu.sync_copy(idx_hbm.at[pl.ds(base, bpw)], idx_v)

    @pl.loop(0, bpw // 16, unroll=8)
    def _xform(i):
        v = idx_v[pl.ds(i * 16, 16)]
        j = lax.bitwise_and(v, _BC - 1)
        hi = lax.shift_right_logical(j, _SHIFT)
        idx_v[pl.ds(i * 16, 16)] = v + j - (_BC - 1) * hi

    def start_gather(c, b):
        pltpu.async_copy(
            table_hbm.at[idx_v.at[pl.ds(c * _CHUNK, _CHUNK)]], rows[b], gsem[b]
        )

    start_gather(0, 0)
    start_gather(1, 1)

    @pl.loop(0, nchunk, step=2)
    def _pair(g):
        for b in range(2):
            c = g + b
            o = 1 - b
            @pl.when(jnp.logical_and(c >= 1, c + 1 < nchunk))
            def _refill():
                pltpu.make_async_copy(
                    rows[o], out_hbm.at[pl.ds(0, _CHUNK), pl.ds(0, _D)], ssem[o]
                ).wait()
                start_gather(c + 1, o)

            pltpu.make_async_copy(
                table_hbm.at[idx_v.at[pl.ds(0, _CHUNK)]], rows[b], gsem[b]
            ).wait()
            pltpu.async_copy(
                rows[b],
                out_hbm.at[pl.ds(base + c * _CHUNK, _CHUNK), pl.ds(0, _D)],
                ssem[b],
            )

    for b in range(2):
        pltpu.make_async_copy(
            rows[b], out_hbm.at[pl.ds(0, _CHUNK), pl.ds(0, _D)], ssem[b]
        ).wait()


def kernel(input, weight):
    b = input.size
    flat_idx = input.reshape(b).astype(jnp.int32)
    wp = _transpose_table(weight.T)
    wd = wp.reshape(2 * wp.shape[0], _D)
    bpw = b // _NW
    k = pl.kernel(
        _gather_body,
        out_type=jax.ShapeDtypeStruct((b, 2 * _D), jnp.float32),
        mesh=plsc.VectorSubcoreMesh(core_axis_name="c", subcore_axis_name="s"),
        scratch_types=[
            pltpu.VMEM((bpw,), jnp.int32),
            pltpu.VMEM((_CHUNK, _D), jnp.float32),
            pltpu.VMEM((_CHUNK, _D), jnp.float32),
            pltpu.SemaphoreType.DMA,
            pltpu.SemaphoreType.DMA,
            pltpu.SemaphoreType.DMA,
            pltpu.SemaphoreType.DMA,
        ],
        compiler_params=pltpu.CompilerParams(use_tc_tiling_on_sc=False),
    )
    out = k(flat_idx, wd)
    return out[:, :_D].reshape(*input.shape, _D)
